# baseline (speedup 1.0000x reference)
import functools

import jax
import jax.numpy as jnp
from jax import lax
from jax.experimental import pallas as pl
from jax.experimental.pallas import tpu as pltpu
from jax.experimental.pallas import tpu_sc as plsc

_B, _N, _F, _C = 4, 400, 240, 32
_R_TC = 336
_R_SC = _N - _R_TC
_NW = 32
_WPB = _NW // _B
_RPW = _R_SC // _WPB
_NJB = _N // 16


def _pre_body(x_ref, wsi_ref, wsj_ref, bs_ref, u_ref, vt_ref):
    f32 = jnp.float32
    xb = x_ref[0]
    u_ref[0] = jnp.dot(xb, wsi_ref[...], preferred_element_type=f32) \
        + bs_ref[...]
    vt_ref[0] = jnp.dot(xb, wsj_ref[...], preferred_element_type=f32).T


def _pool_tc_body(u_ref, vt_ref, a_ref, e_ref, et_ref, wsc_ref,
                  pi_ref, pjt_ref, s_scr, di_scr, dj_scr):
    R = _R_TC
    N = _N
    C = _C
    f32 = jnp.float32

    ones_col = jnp.ones((N, 1), f32)
    ones_row = jnp.ones((1, R), f32)
    iota_cc = lax.broadcasted_iota(jnp.int32, (C, 4), 0)
    iota_4c = lax.broadcasted_iota(jnp.int32, (4, C), 1)
    four_col = lax.broadcasted_iota(jnp.int32, (C, 4), 1)
    four_row = lax.broadcasted_iota(jnp.int32, (4, C), 0)

    di_scr[...] = jnp.zeros((R, N), f32)
    dj_scr[...] = jnp.zeros((R, N), f32)

    def pass1(t, _):
        c0 = 4 * t
        eb = e_ref[0]
        etb = et_ref[0]
        ab = a_ref[0]
        oh4 = (iota_cc == c0 + four_col).astype(f32)
        ucols = jnp.dot(u_ref[0], oh4, preferred_element_type=f32)
        sv = []
        for k in range(4):
            ub = jnp.broadcast_to(ucols[:, k:k + 1], (R, N))
            vb = jnp.broadcast_to(vt_ref[0, pl.ds(c0 + k, 1), :], (R, N))
            s_k = jnp.maximum(ub + vb + wsc_ref[0, c0 + k] * eb
                              + wsc_ref[1, c0 + k] * etb, 0.0) * ab
            s_scr[pl.ds(c0 + k, 1), :, :] = s_k[None]
            sv.append(s_k)
        di_scr[...] = di_scr[...] + (
            (wsc_ref[2, c0] * sv[0] + wsc_ref[2, c0 + 1] * sv[1])
            + (wsc_ref[2, c0 + 2] * sv[2] + wsc_ref[2, c0 + 3] * sv[3]))
        dj_scr[...] = dj_scr[...] + (
            (wsc_ref[3, c0] * sv[0] + wsc_ref[3, c0 + 1] * sv[1])
            + (wsc_ref[3, c0 + 2] * sv[2] + wsc_ref[3, c0 + 3] * sv[3]))
        return 0

    lax.fori_loop(0, C // 4, pass1, 0)

    di_scr[...] = jax.nn.sigmoid(di_scr[...] + wsc_ref[4, 0])
    dj_scr[...] = jax.nn.sigmoid(dj_scr[...] + wsc_ref[4, 1])

    pi_ref[0] = jnp.zeros((R, C), f32)
    pjt_ref[0] = jnp.zeros((C, N), f32)

    def pass2(t, _):
        c0 = 4 * t
        sigi = di_scr[...]
        sigj = dj_scr[...]
        pcl = []
        prl = []
        for k in range(4):
            s_k = s_scr[c0 + k]
            pcl.append(jnp.dot(s_k * sigi, ones_col,
                               preferred_element_type=f32))
            prl.append(jnp.dot(ones_row, s_k * sigj,
                               preferred_element_type=f32))
        pcols = jnp.concatenate(pcl, axis=1)
        prows = jnp.concatenate(prl, axis=0)
        ohrows = (iota_4c == c0 + four_row).astype(f32)
        pi_ref[0] = pi_ref[0] + jnp.dot(pcols, ohrows,
                                        preferred_element_type=f32)
        pjt_ref[0] = pjt_ref[0] + lax.dot_general(
            ohrows, prows, (((0,), (0,)), ((), ())),
            preferred_element_type=f32)
        return 0

    lax.fori_loop(0, C // 4, pass2, 0)


def _pool_sc_body(vt_hbm, e_hbm, et_hbm, a_hbm, usp_hbm, w_hbm,
                  pip_hbm, pjp_hbm,
                  vt_v, e_v, et_v, a_v, usp_v, w_v, s_v, di_v, dj_v,
                  pib_v, pjl_v):
    f32 = jnp.float32
    wid = lax.axis_index("s") * 2 + lax.axis_index("c")
    b = wid // _WPB
    k = wid % _WPB
    i0 = _R_TC + k * _RPW

    pltpu.sync_copy(vt_hbm.at[pl.ds(b * _C * _N, _C * _N)], vt_v)
    row_off = b * _N * _N + i0 * _N
    slab = _RPW * _N
    pltpu.sync_copy(e_hbm.at[pl.ds(row_off, slab)], e_v)
    pltpu.sync_copy(et_hbm.at[pl.ds(row_off, slab)], et_v)
    pltpu.sync_copy(a_hbm.at[pl.ds(row_off, slab)], a_v)
    pltpu.sync_copy(usp_hbm.at[pl.ds((b * _R_SC + k * _RPW) * _C * 16,
                                     _RPW * _C * 16)], usp_v)
    pltpu.sync_copy(w_hbm.at[pl.ds(0, (4 * _C + 2) * 16)], w_v)

    zero16 = jnp.zeros((16,), f32)

    def zero_pjl(m, _):
        pjl_v[pl.ds(m * 16, 16)] = zero16
        return 0

    lax.fori_loop(0, _C * _N // 16, zero_pjl, 0)

    def zero_pib(m, _):
        pib_v[pl.ds(m * 16, 16)] = zero16
        return 0

    lax.fori_loop(0, _RPW * _C, zero_pib, 0)

    bai_v = w_v[pl.ds(4 * _C * 16, 16)]
    baj_v = w_v[pl.ds(4 * _C * 16 + 16, 16)]

    def row_body(r, _):
        rbase = r * _N

        def jb_a(jb, _a):
            base = rbase + jb * 16
            ev = e_v[pl.ds(base, 16)]
            etv = et_v[pl.ds(base, 16)]
            av = a_v[pl.ds(base, 16)]
            di = zero16
            dj = zero16
            for c in range(_C):
                off = c * _N + jb * 16
                pre = (usp_v[pl.ds(r * _C * 16 + c * 16, 16)]
                       + vt_v[pl.ds(off, 16)]
                       + w_v[pl.ds(c * 16, 16)] * ev
                       + w_v[pl.ds(_C * 16 + c * 16, 16)] * etv)
                s = jnp.maximum(pre, 0.0) * av
                s_v[pl.ds(off, 16)] = s
                di = di + w_v[pl.ds(2 * _C * 16 + c * 16, 16)] * s
                dj = dj + w_v[pl.ds(3 * _C * 16 + c * 16, 16)] * s
            di_v[pl.ds(jb * 16, 16)] = di
            dj_v[pl.ds(jb * 16, 16)] = dj
            return 0

        lax.fori_loop(0, _NJB, jb_a, 0)

        def jb_b(jb, _b):
            zi = di_v[pl.ds(jb * 16, 16)] + bai_v
            zj = dj_v[pl.ds(jb * 16, 16)] + baj_v
            sigi = 1.0 / (1.0 + jnp.exp(-zi))
            sigj = 1.0 / (1.0 + jnp.exp(-zj))
            for c in range(_C):
                off = c * _N + jb * 16
                s = s_v[pl.ds(off, 16)]
                poff = r * _C * 16 + c * 16
                pib_v[pl.ds(poff, 16)] = pib_v[pl.ds(poff, 16)] + s * sigi
                pjl_v[pl.ds(off, 16)] = pjl_v[pl.ds(off, 16)] + s * sigj
            return 0

        lax.fori_loop(0, _NJB, jb_b, 0)
        return 0

    lax.fori_loop(0, _RPW, row_body, 0)

    pltpu.sync_copy(pib_v, pip_hbm.at[pl.ds(wid * _RPW * _C * 16,
                                            _RPW * _C * 16)])
    pltpu.sync_copy(pjl_v, pjp_hbm.at[pl.ds(wid * _C * _N, _C * _N)])


def _post_body(x_ref, pi_tc_ref, pi_sc_ref, pjt_tc_ref, pjp_ref,
               wnx_ref, wni_ref, wnj_ref, bn_ref, wd_ref, bd_ref, out_ref):
    f32 = jnp.float32
    xb = x_ref[0]
    pool_i = jnp.concatenate([pi_tc_ref[0], pi_sc_ref[0]], axis=0)
    pjt = pjt_tc_ref[0] + jnp.sum(pjp_ref[0], axis=0)
    xo = (jnp.dot(xb, wnx_ref[...], preferred_element_type=f32)
          + jnp.dot(pool_i, wni_ref[...], preferred_element_type=f32)
          + lax.dot_general(pjt, wnj_ref[...],
                            (((0,), (0,)), ((), ())),
                            preferred_element_type=f32)
          + bn_ref[...])
    out_ref[0] = jnp.dot(xo, wd_ref[...], preferred_element_type=f32) \
        + bd_ref[...]


def kernel(x, a, e, Ws, bs, Wai, bai, Waj, baj, Wn, bn, We, be, Wd, bd):
    B, N, F, C = _B, _N, _F, _C
    LBL = Wd.shape[1]
    f32 = jnp.float32

    e2 = e[..., 0]
    et2 = jnp.swapaxes(e2, 1, 2)
    wsi = Ws[:F]
    wsj = Ws[F:2 * F]
    brow = jnp.zeros((C,), f32).at[0].set(bai[0]).at[1].set(baj[0])
    wsc = jnp.stack([Ws[2 * F], Ws[2 * F + 1], Wai[:, 0], Waj[:, 0], brow],
                    axis=0)
    wnx = Wn[:F]
    wni = Wn[F:F + C]
    wnj = Wn[F + C:]

    u, vt = pl.pallas_call(
        _pre_body,
        grid=(B,),
        in_specs=[
            pl.BlockSpec((1, N, F), lambda b: (b, 0, 0)),
            pl.BlockSpec((F, C), lambda b: (0, 0)),
            pl.BlockSpec((F, C), lambda b: (0, 0)),
            pl.BlockSpec((1, C), lambda b: (0, 0)),
        ],
        out_specs=[pl.BlockSpec((1, N, C), lambda b: (b, 0, 0)),
                   pl.BlockSpec((1, C, N), lambda b: (b, 0, 0))],
        out_shape=[jax.ShapeDtypeStruct((B, N, C), f32),
                   jax.ShapeDtypeStruct((B, C, N), f32)],
    )(x, wsi, wsj, bs[None])

    pi_tc, pjt_tc = pl.pallas_call(
        _pool_tc_body,
        grid=(B,),
        in_specs=[
            pl.BlockSpec((1, _R_TC, C), lambda b: (b, 0, 0)),
            pl.BlockSpec((1, C, N), lambda b: (b, 0, 0)),
            pl.BlockSpec((1, _R_TC, N), lambda b: (b, 0, 0)),
            pl.BlockSpec((1, _R_TC, N), lambda b: (b, 0, 0)),
            pl.BlockSpec((1, _R_TC, N), lambda b: (b, 0, 0)),
            pl.BlockSpec(memory_space=pltpu.SMEM),
        ],
        out_specs=[pl.BlockSpec((1, _R_TC, C), lambda b: (b, 0, 0)),
                   pl.BlockSpec((1, C, N), lambda b: (b, 0, 0))],
        out_shape=[jax.ShapeDtypeStruct((B, _R_TC, C), f32),
                   jax.ShapeDtypeStruct((B, C, N), f32)],
        scratch_shapes=[
            pltpu.VMEM((C, _R_TC, N), f32),
            pltpu.VMEM((_R_TC, N), f32),
            pltpu.VMEM((_R_TC, N), f32),
        ],
    )(u, vt, a, e2, et2, wsc)

    u_sc = u[:, _R_TC:, :]
    usp = jnp.broadcast_to(u_sc[..., None],
                           (B, _R_SC, C, 16)).reshape(-1)
    wflat = jnp.concatenate([Ws[2 * F], Ws[2 * F + 1], Wai[:, 0],
                             Waj[:, 0], bai, baj])
    wspl = jnp.broadcast_to(wflat[:, None], (4 * C + 2, 16)).reshape(-1)

    sc_pool = functools.partial(
        pl.kernel,
        mesh=plsc.VectorSubcoreMesh(core_axis_name="c",
                                    subcore_axis_name="s"),
        out_type=[jax.ShapeDtypeStruct((_NW * _RPW * C * 16,), f32),
                  jax.ShapeDtypeStruct((_NW * C * N,), f32)],
        scratch_types=[
            pltpu.VMEM((C * N,), f32),
            pltpu.VMEM((_RPW * N,), f32),
            pltpu.VMEM((_RPW * N,), f32),
            pltpu.VMEM((_RPW * N,), f32),
            pltpu.VMEM((_RPW * C * 16,), f32),
            pltpu.VMEM(((4 * C + 2) * 16,), f32),
            pltpu.VMEM((C * N,), f32),
            pltpu.VMEM((N,), f32),
            pltpu.VMEM((N,), f32),
            pltpu.VMEM((_RPW * C * 16,), f32),
            pltpu.VMEM((C * N,), f32),
        ],
    )(_pool_sc_body)
    pip, pjp = sc_pool(vt.reshape(-1), e2.reshape(-1), et2.reshape(-1),
                       a.reshape(-1), usp, wspl)

    pi_sc = jnp.sum(pip.reshape(B, _R_SC, C, 16), axis=-1)
    pjp4 = pjp.reshape(B, _WPB, C, N)

    out = pl.pallas_call(
        _post_body,
        grid=(B,),
        in_specs=[
            pl.BlockSpec((1, N, F), lambda b: (b, 0, 0)),
            pl.BlockSpec((1, _R_TC, C), lambda b: (b, 0, 0)),
            pl.BlockSpec((1, _R_SC, C), lambda b: (b, 0, 0)),
            pl.BlockSpec((1, C, N), lambda b: (b, 0, 0)),
            pl.BlockSpec((1, _WPB, C, N), lambda b: (b, 0, 0, 0)),
            pl.BlockSpec((F, F), lambda b: (0, 0)),
            pl.BlockSpec((C, F), lambda b: (0, 0)),
            pl.BlockSpec((C, F), lambda b: (0, 0)),
            pl.BlockSpec((1, F), lambda b: (0, 0)),
            pl.BlockSpec((F, LBL), lambda b: (0, 0)),
            pl.BlockSpec((1, LBL), lambda b: (0, 0)),
        ],
        out_specs=pl.BlockSpec((1, N, LBL), lambda b: (b, 0, 0)),
        out_shape=jax.ShapeDtypeStruct((B, N, LBL), f32),
    )(x, pi_tc, pi_sc, pjt_tc, pjp4, wnx, wni, wnj, bn[None], Wd, bd[None])
    return out


# speedup vs baseline: 1.4272x; 1.4272x over previous
import functools

import jax
import jax.numpy as jnp
from jax import lax
from jax.experimental import pallas as pl
from jax.experimental.pallas import tpu as pltpu
from jax.experimental.pallas import tpu_sc as plsc

_B, _N, _F, _C = 4, 400, 240, 32
_R_TC = 368
_R_SC = _N - _R_TC
_NW = 32
_WPB = _NW // _B
_RPW = _R_SC // _WPB
_NJB = _N // 16


def _pre_body(x_ref, wsi_ref, wsj_ref, bs_ref, u_ref, vt_ref):
    f32 = jnp.float32
    xb = x_ref[0]
    u_ref[0] = jnp.dot(xb, wsi_ref[...], preferred_element_type=f32) \
        + bs_ref[...]
    vt_ref[0] = jnp.dot(xb, wsj_ref[...], preferred_element_type=f32).T


def _pool_tc_body(u_ref, vt_ref, a_ref, e_ref, et_ref, wsc_ref,
                  pi_ref, pjt_ref, s_scr, di_scr, dj_scr):
    R = _R_TC
    N = _N
    C = _C
    f32 = jnp.float32

    ones_col = jnp.ones((N, 1), f32)
    ones_row = jnp.ones((1, R), f32)
    iota_cc = lax.broadcasted_iota(jnp.int32, (C, 4), 0)
    iota_4c = lax.broadcasted_iota(jnp.int32, (4, C), 1)
    four_col = lax.broadcasted_iota(jnp.int32, (C, 4), 1)
    four_row = lax.broadcasted_iota(jnp.int32, (4, C), 0)

    di_scr[...] = jnp.zeros((R, N), f32)
    dj_scr[...] = jnp.zeros((R, N), f32)

    def pass1(t, _):
        c0 = 4 * t
        eb = e_ref[0]
        etb = et_ref[0]
        ab = a_ref[0]
        oh4 = (iota_cc == c0 + four_col).astype(f32)
        ucols = jnp.dot(u_ref[0], oh4, preferred_element_type=f32)
        sv = []
        for k in range(4):
            ub = jnp.broadcast_to(ucols[:, k:k + 1], (R, N))
            vb = jnp.broadcast_to(vt_ref[0, pl.ds(c0 + k, 1), :], (R, N))
            s_k = jnp.maximum(ub + vb + wsc_ref[0, c0 + k] * eb
                              + wsc_ref[1, c0 + k] * etb, 0.0) * ab
            s_scr[pl.ds(c0 + k, 1), :, :] = s_k[None]
            sv.append(s_k)
        di_scr[...] = di_scr[...] + (
            (wsc_ref[2, c0] * sv[0] + wsc_ref[2, c0 + 1] * sv[1])
            + (wsc_ref[2, c0 + 2] * sv[2] + wsc_ref[2, c0 + 3] * sv[3]))
        dj_scr[...] = dj_scr[...] + (
            (wsc_ref[3, c0] * sv[0] + wsc_ref[3, c0 + 1] * sv[1])
            + (wsc_ref[3, c0 + 2] * sv[2] + wsc_ref[3, c0 + 3] * sv[3]))
        return 0

    lax.fori_loop(0, C // 4, pass1, 0)

    di_scr[...] = jax.nn.sigmoid(di_scr[...] + wsc_ref[4, 0])
    dj_scr[...] = jax.nn.sigmoid(dj_scr[...] + wsc_ref[4, 1])

    pi_ref[0] = jnp.zeros((R, C), f32)
    pjt_ref[0] = jnp.zeros((C, N), f32)

    def pass2(t, _):
        c0 = 4 * t
        sigi = di_scr[...]
        sigj = dj_scr[...]
        pcl = []
        prl = []
        for k in range(4):
            s_k = s_scr[c0 + k]
            pcl.append(jnp.dot(s_k * sigi, ones_col,
                               preferred_element_type=f32))
            prl.append(jnp.dot(ones_row, s_k * sigj,
                               preferred_element_type=f32))
        pcols = jnp.concatenate(pcl, axis=1)
        prows = jnp.concatenate(prl, axis=0)
        ohrows = (iota_4c == c0 + four_row).astype(f32)
        pi_ref[0] = pi_ref[0] + jnp.dot(pcols, ohrows,
                                        preferred_element_type=f32)
        pjt_ref[0] = pjt_ref[0] + lax.dot_general(
            ohrows, prows, (((0,), (0,)), ((), ())),
            preferred_element_type=f32)
        return 0

    lax.fori_loop(0, C // 4, pass2, 0)


def _pool_sc_body(vt_hbm, e_hbm, et_hbm, a_hbm, usp_hbm, w_hbm,
                  pip_hbm, pjp_hbm,
                  vt_v, e_v, et_v, a_v, usp_v, w_v, s_v, di_v, dj_v,
                  pib_v, pjl_v):
    f32 = jnp.float32
    wid = lax.axis_index("s") * 2 + lax.axis_index("c")
    b = wid // _WPB
    k = wid % _WPB
    i0 = _R_TC + k * _RPW

    pltpu.sync_copy(vt_hbm.at[pl.ds(b * _C * _N, _C * _N)], vt_v)
    row_off = b * _N * _N + i0 * _N
    slab = _RPW * _N
    pltpu.sync_copy(e_hbm.at[pl.ds(row_off, slab)], e_v)
    pltpu.sync_copy(et_hbm.at[pl.ds(row_off, slab)], et_v)
    pltpu.sync_copy(a_hbm.at[pl.ds(row_off, slab)], a_v)
    pltpu.sync_copy(usp_hbm.at[pl.ds((b * _R_SC + k * _RPW) * _C * 16,
                                     _RPW * _C * 16)], usp_v)
    pltpu.sync_copy(w_hbm.at[pl.ds(0, (4 * _C + 2) * 16)], w_v)

    zero16 = jnp.zeros((16,), f32)

    def zero_pjl(m, _):
        pjl_v[pl.ds(m * 16, 16)] = zero16
        return 0

    lax.fori_loop(0, _C * _N // 16, zero_pjl, 0)

    def zero_pib(m, _):
        pib_v[pl.ds(m * 16, 16)] = zero16
        return 0

    lax.fori_loop(0, _RPW * _C, zero_pib, 0)

    bai_v = w_v[pl.ds(4 * _C * 16, 16)]
    baj_v = w_v[pl.ds(4 * _C * 16 + 16, 16)]

    def row_body(r, _):
        rbase = r * _N

        def jb_a(jb, _a):
            base = rbase + jb * 16
            ev = e_v[pl.ds(base, 16)]
            etv = et_v[pl.ds(base, 16)]
            av = a_v[pl.ds(base, 16)]
            di = zero16
            dj = zero16
            for c in range(_C):
                off = c * _N + jb * 16
                pre = (usp_v[pl.ds(r * _C * 16 + c * 16, 16)]
                       + vt_v[pl.ds(off, 16)]
                       + w_v[pl.ds(c * 16, 16)] * ev
                       + w_v[pl.ds(_C * 16 + c * 16, 16)] * etv)
                s = jnp.maximum(pre, 0.0) * av
                s_v[pl.ds(off, 16)] = s
                di = di + w_v[pl.ds(2 * _C * 16 + c * 16, 16)] * s
                dj = dj + w_v[pl.ds(3 * _C * 16 + c * 16, 16)] * s
            di_v[pl.ds(jb * 16, 16)] = di
            dj_v[pl.ds(jb * 16, 16)] = dj
            return 0

        lax.fori_loop(0, _NJB, jb_a, 0)

        def jb_b(jb, _b):
            zi = di_v[pl.ds(jb * 16, 16)] + bai_v
            zj = dj_v[pl.ds(jb * 16, 16)] + baj_v
            sigi = 1.0 / (1.0 + jnp.exp(-zi))
            sigj = 1.0 / (1.0 + jnp.exp(-zj))
            for c in range(_C):
                off = c * _N + jb * 16
                s = s_v[pl.ds(off, 16)]
                poff = r * _C * 16 + c * 16
                pib_v[pl.ds(poff, 16)] = pib_v[pl.ds(poff, 16)] + s * sigi
                pjl_v[pl.ds(off, 16)] = pjl_v[pl.ds(off, 16)] + s * sigj
            return 0

        lax.fori_loop(0, _NJB, jb_b, 0)
        return 0

    lax.fori_loop(0, _RPW, row_body, 0)

    pltpu.sync_copy(pib_v, pip_hbm.at[pl.ds(wid * _RPW * _C * 16,
                                            _RPW * _C * 16)])
    pltpu.sync_copy(pjl_v, pjp_hbm.at[pl.ds(wid * _C * _N, _C * _N)])


def _post_body(x_ref, pi_tc_ref, pi_sc_ref, pjt_tc_ref, pjp_ref,
               wnx_ref, wni_ref, wnj_ref, bn_ref, wd_ref, bd_ref, out_ref):
    f32 = jnp.float32
    xb = x_ref[0]
    pool_i = jnp.concatenate([pi_tc_ref[0], pi_sc_ref[0]], axis=0)
    pjt = pjt_tc_ref[0] + jnp.sum(pjp_ref[0], axis=0)
    xo = (jnp.dot(xb, wnx_ref[...], preferred_element_type=f32)
          + jnp.dot(pool_i, wni_ref[...], preferred_element_type=f32)
          + lax.dot_general(pjt, wnj_ref[...],
                            (((0,), (0,)), ((), ())),
                            preferred_element_type=f32)
          + bn_ref[...])
    out_ref[0] = jnp.dot(xo, wd_ref[...], preferred_element_type=f32) \
        + bd_ref[...]


def kernel(x, a, e, Ws, bs, Wai, bai, Waj, baj, Wn, bn, We, be, Wd, bd):
    B, N, F, C = _B, _N, _F, _C
    LBL = Wd.shape[1]
    f32 = jnp.float32

    e2 = e[..., 0]
    et2 = jnp.swapaxes(e2, 1, 2)
    wsi = Ws[:F]
    wsj = Ws[F:2 * F]
    brow = jnp.zeros((C,), f32).at[0].set(bai[0]).at[1].set(baj[0])
    wsc = jnp.stack([Ws[2 * F], Ws[2 * F + 1], Wai[:, 0], Waj[:, 0], brow],
                    axis=0)
    wnx = Wn[:F]
    wni = Wn[F:F + C]
    wnj = Wn[F + C:]

    u, vt = pl.pallas_call(
        _pre_body,
        grid=(B,),
        in_specs=[
            pl.BlockSpec((1, N, F), lambda b: (b, 0, 0)),
            pl.BlockSpec((F, C), lambda b: (0, 0)),
            pl.BlockSpec((F, C), lambda b: (0, 0)),
            pl.BlockSpec((1, C), lambda b: (0, 0)),
        ],
        out_specs=[pl.BlockSpec((1, N, C), lambda b: (b, 0, 0)),
                   pl.BlockSpec((1, C, N), lambda b: (b, 0, 0))],
        out_shape=[jax.ShapeDtypeStruct((B, N, C), f32),
                   jax.ShapeDtypeStruct((B, C, N), f32)],
    )(x, wsi, wsj, bs[None])

    u_sc = u[:, _R_TC:, :]
    usp = jnp.broadcast_to(u_sc[..., None],
                           (B, _R_SC, C, 16)).reshape(-1)
    wflat = jnp.concatenate([Ws[2 * F], Ws[2 * F + 1], Wai[:, 0],
                             Waj[:, 0], bai, baj])
    wspl = jnp.broadcast_to(wflat[:, None], (4 * C + 2, 16)).reshape(-1)

    sc_pool = functools.partial(
        pl.kernel,
        mesh=plsc.VectorSubcoreMesh(core_axis_name="c",
                                    subcore_axis_name="s"),
        out_type=[jax.ShapeDtypeStruct((_NW * _RPW * C * 16,), f32),
                  jax.ShapeDtypeStruct((_NW * C * N,), f32)],
        scratch_types=[
            pltpu.VMEM((C * N,), f32),
            pltpu.VMEM((_RPW * N,), f32),
            pltpu.VMEM((_RPW * N,), f32),
            pltpu.VMEM((_RPW * N,), f32),
            pltpu.VMEM((_RPW * C * 16,), f32),
            pltpu.VMEM(((4 * C + 2) * 16,), f32),
            pltpu.VMEM((C * N,), f32),
            pltpu.VMEM((N,), f32),
            pltpu.VMEM((N,), f32),
            pltpu.VMEM((_RPW * C * 16,), f32),
            pltpu.VMEM((C * N,), f32),
        ],
    )(_pool_sc_body)
    pip, pjp = sc_pool(vt.reshape(-1), e2.reshape(-1), et2.reshape(-1),
                       a.reshape(-1), usp, wspl)

    pi_tc, pjt_tc = pl.pallas_call(
        _pool_tc_body,
        grid=(B,),
        in_specs=[
            pl.BlockSpec((1, _R_TC, C), lambda b: (b, 0, 0)),
            pl.BlockSpec((1, C, N), lambda b: (b, 0, 0)),
            pl.BlockSpec((1, _R_TC, N), lambda b: (b, 0, 0)),
            pl.BlockSpec((1, _R_TC, N), lambda b: (b, 0, 0)),
            pl.BlockSpec((1, _R_TC, N), lambda b: (b, 0, 0)),
            pl.BlockSpec(memory_space=pltpu.SMEM),
        ],
        out_specs=[pl.BlockSpec((1, _R_TC, C), lambda b: (b, 0, 0)),
                   pl.BlockSpec((1, C, N), lambda b: (b, 0, 0))],
        out_shape=[jax.ShapeDtypeStruct((B, _R_TC, C), f32),
                   jax.ShapeDtypeStruct((B, C, N), f32)],
        scratch_shapes=[
            pltpu.VMEM((C, _R_TC, N), f32),
            pltpu.VMEM((_R_TC, N), f32),
            pltpu.VMEM((_R_TC, N), f32),
        ],
    )(u, vt, a, e2, et2, wsc)

    pi_sc = jnp.sum(pip.reshape(B, _R_SC, C, 16), axis=-1)
    pjp4 = pjp.reshape(B, _WPB, C, N)

    out = pl.pallas_call(
        _post_body,
        grid=(B,),
        in_specs=[
            pl.BlockSpec((1, N, F), lambda b: (b, 0, 0)),
            pl.BlockSpec((1, _R_TC, C), lambda b: (b, 0, 0)),
            pl.BlockSpec((1, _R_SC, C), lambda b: (b, 0, 0)),
            pl.BlockSpec((1, C, N), lambda b: (b, 0, 0)),
            pl.BlockSpec((1, _WPB, C, N), lambda b: (b, 0, 0, 0)),
            pl.BlockSpec((F, F), lambda b: (0, 0)),
            pl.BlockSpec((C, F), lambda b: (0, 0)),
            pl.BlockSpec((C, F), lambda b: (0, 0)),
            pl.BlockSpec((1, F), lambda b: (0, 0)),
            pl.BlockSpec((F, LBL), lambda b: (0, 0)),
            pl.BlockSpec((1, LBL), lambda b: (0, 0)),
        ],
        out_specs=pl.BlockSpec((1, N, LBL), lambda b: (b, 0, 0)),
        out_shape=jax.ShapeDtypeStruct((B, N, LBL), f32),
    )(x, pi_tc, pi_sc, pjt_tc, pjp4, wnx, wni, wnj, bn[None], Wd, bd[None])
    return out


# speedup vs baseline: 2.3191x; 1.6249x over previous
import jax
import jax.numpy as jnp
from jax import lax
from jax.experimental import pallas as pl
from jax.experimental.pallas import tpu as pltpu


def _net_body(x_ref, a_ref, e_ref, et_ref, wsi_ref, wsj_ref, bs_ref,
              wsc_ref, wnx_ref, wni_ref, wnj_ref, bn_ref, wd_ref, bd_ref,
              out_ref, s_scr, vt_scr, u_scr, di_scr, dj_scr, pi_scr, pjt_scr):
    N = a_ref.shape[1]
    C = bs_ref.shape[1]
    f32 = jnp.float32

    xb = x_ref[0]

    u_scr[...] = jnp.dot(xb, wsi_ref[...],
                         preferred_element_type=f32) + bs_ref[...]
    v = jnp.dot(xb, wsj_ref[...], preferred_element_type=f32)
    vt_scr[...] = v.T

    ones_col = jnp.ones((N, 1), f32)
    ones_row = jnp.ones((1, N), f32)
    iota_cc = lax.broadcasted_iota(jnp.int32, (C, 8), 0)
    iota_8c = lax.broadcasted_iota(jnp.int32, (8, C), 1)
    eight_col = lax.broadcasted_iota(jnp.int32, (C, 8), 1)
    eight_row = lax.broadcasted_iota(jnp.int32, (8, C), 0)

    di_scr[...] = jnp.zeros((N, N), f32)
    dj_scr[...] = jnp.zeros((N, N), f32)

    def pass1(t, _):
        c0 = 8 * t
        eb = e_ref[0]
        etb = et_ref[0]
        ab = a_ref[0]
        oh8 = (iota_cc == c0 + eight_col).astype(f32)
        ucols = jnp.dot(u_scr[...], oh8, preferred_element_type=f32)
        sv = []
        for k in range(8):
            ub = jnp.broadcast_to(ucols[:, k:k + 1], (N, N))
            vb = jnp.broadcast_to(vt_scr[pl.ds(c0 + k, 1), :], (N, N))
            s_k = jnp.maximum(ub + vb + wsc_ref[0, c0 + k] * eb
                              + wsc_ref[1, c0 + k] * etb, 0.0) * ab
            s_scr[pl.ds(c0 + k, 1), :, :] = s_k[None]
            sv.append(s_k)
        acc_i = wsc_ref[2, c0] * sv[0] + wsc_ref[2, c0 + 1] * sv[1]
        acc_j = wsc_ref[3, c0] * sv[0] + wsc_ref[3, c0 + 1] * sv[1]
        for k in range(2, 8):
            acc_i = acc_i + wsc_ref[2, c0 + k] * sv[k]
            acc_j = acc_j + wsc_ref[3, c0 + k] * sv[k]
        di_scr[...] = di_scr[...] + acc_i
        dj_scr[...] = dj_scr[...] + acc_j
        return 0

    lax.fori_loop(0, C // 8, pass1, 0)

    bai = wsc_ref[4, 0]
    baj = wsc_ref[4, 1]
    di_scr[...] = jax.nn.sigmoid(di_scr[...] + bai)
    dj_scr[...] = jax.nn.sigmoid(dj_scr[...] + baj)

    pi_scr[...] = jnp.zeros((N, C), f32)
    pjt_scr[...] = jnp.zeros((C, N), f32)

    def pass2(t, _):
        c0 = 8 * t
        sigi = di_scr[...]
        sigj = dj_scr[...]
        pcl = []
        prl = []
        for k in range(8):
            s_k = s_scr[c0 + k]
            pcl.append(jnp.dot(s_k * sigi, ones_col,
                               preferred_element_type=f32))
            prl.append(jnp.dot(ones_row, s_k * sigj,
                               preferred_element_type=f32))
        pcols = jnp.concatenate(pcl, axis=1)
        prows = jnp.concatenate(prl, axis=0)
        ohrows = (iota_8c == c0 + eight_row).astype(f32)
        pi_scr[...] = pi_scr[...] + jnp.dot(pcols, ohrows,
                                            preferred_element_type=f32)
        pjt_scr[...] = pjt_scr[...] + lax.dot_general(
            ohrows, prows, (((0,), (0,)), ((), ())),
            preferred_element_type=f32)
        return 0

    lax.fori_loop(0, C // 8, pass2, 0)

    xo = (jnp.dot(xb, wnx_ref[...], preferred_element_type=f32)
          + jnp.dot(pi_scr[...], wni_ref[...], preferred_element_type=f32)
          + lax.dot_general(pjt_scr[...], wnj_ref[...],
                            (((0,), (0,)), ((), ())),
                            preferred_element_type=f32)
          + bn_ref[...])
    out_ref[0] = jnp.dot(xo, wd_ref[...], preferred_element_type=f32) \
        + bd_ref[...]


def kernel(x, a, e, Ws, bs, Wai, bai, Waj, baj, Wn, bn, We, be, Wd, bd):
    B, N, F = x.shape
    C = Ws.shape[1]
    LBL = Wd.shape[1]
    f32 = jnp.float32

    e2 = e[..., 0]
    et2 = jnp.swapaxes(e2, 1, 2)
    wsi = Ws[:F]
    wsj = Ws[F:2 * F]
    brow = jnp.zeros((C,), f32).at[0].set(bai[0]).at[1].set(baj[0])
    wsc = jnp.stack([Ws[2 * F], Ws[2 * F + 1], Wai[:, 0], Waj[:, 0], brow],
                    axis=0)
    wnx = Wn[:F]
    wni = Wn[F:F + C]
    wnj = Wn[F + C:]

    out = pl.pallas_call(
        _net_body,
        grid=(B,),
        in_specs=[
            pl.BlockSpec((1, N, F), lambda b: (b, 0, 0)),
            pl.BlockSpec((1, N, N), lambda b: (b, 0, 0)),
            pl.BlockSpec((1, N, N), lambda b: (b, 0, 0)),
            pl.BlockSpec((1, N, N), lambda b: (b, 0, 0)),
            pl.BlockSpec((F, C), lambda b: (0, 0)),
            pl.BlockSpec((F, C), lambda b: (0, 0)),
            pl.BlockSpec((1, C), lambda b: (0, 0)),
            pl.BlockSpec(memory_space=pltpu.SMEM),
            pl.BlockSpec((F, F), lambda b: (0, 0)),
            pl.BlockSpec((C, F), lambda b: (0, 0)),
            pl.BlockSpec((C, F), lambda b: (0, 0)),
            pl.BlockSpec((1, F), lambda b: (0, 0)),
            pl.BlockSpec((F, LBL), lambda b: (0, 0)),
            pl.BlockSpec((1, LBL), lambda b: (0, 0)),
        ],
        out_specs=pl.BlockSpec((1, N, LBL), lambda b: (b, 0, 0)),
        out_shape=jax.ShapeDtypeStruct((B, N, LBL), f32),
        scratch_shapes=[
            pltpu.VMEM((C, N, N), f32),
            pltpu.VMEM((C, N), f32),
            pltpu.VMEM((N, C), f32),
            pltpu.VMEM((N, N), f32),
            pltpu.VMEM((N, N), f32),
            pltpu.VMEM((N, C), f32),
            pltpu.VMEM((C, N), f32),
        ],
    )(x, a, e2, et2, wsi, wsj, bs[None], wsc, wnx, wni, wnj,
      bn[None], Wd, bd[None])
    return out


# speedup vs baseline: 2.4345x; 1.0498x over previous
import jax
import jax.numpy as jnp
from jax import lax
from jax.experimental import pallas as pl
from jax.experimental.pallas import tpu as pltpu


def _net_body(x_ref, a_ref, e_ref, et_ref, wsi_ref, wsj_ref, bs_ref,
              wsc_ref, wnx_ref, wni_ref, wnj_ref, bn_ref, wd_ref, bd_ref,
              out_ref, s_scr, vt_scr, u_scr, di_scr, dj_scr, pi_scr, pjt_scr):
    N = a_ref.shape[1]
    C = bs_ref.shape[1]
    f32 = jnp.float32

    xb = x_ref[0]

    u_scr[...] = jnp.dot(xb, wsi_ref[...],
                         preferred_element_type=f32) + bs_ref[...]
    v = jnp.dot(xb, wsj_ref[...], preferred_element_type=f32)
    vt_scr[...] = v.T

    ones_col = jnp.ones((N, 1), f32)
    ones_row = jnp.ones((1, N), f32)
    iota_cc = lax.broadcasted_iota(jnp.int32, (C, 16), 0)
    iota_8c = lax.broadcasted_iota(jnp.int32, (16, C), 1)
    eight_col = lax.broadcasted_iota(jnp.int32, (C, 16), 1)
    eight_row = lax.broadcasted_iota(jnp.int32, (16, C), 0)

    di_scr[...] = jnp.zeros((N, N), f32)
    dj_scr[...] = jnp.zeros((N, N), f32)

    def pass1(t, _):
        c0 = 16 * t
        eb = e_ref[0]
        etb = et_ref[0]
        ab = a_ref[0]
        oh8 = (iota_cc == c0 + eight_col).astype(f32)
        ucols = jnp.dot(u_scr[...], oh8, preferred_element_type=f32)
        sv = []
        for k in range(16):
            ub = jnp.broadcast_to(ucols[:, k:k + 1], (N, N))
            vb = jnp.broadcast_to(vt_scr[pl.ds(c0 + k, 1), :], (N, N))
            s_k = jnp.maximum(ub + vb + wsc_ref[0, c0 + k] * eb
                              + wsc_ref[1, c0 + k] * etb, 0.0) * ab
            s_scr[pl.ds(c0 + k, 1), :, :] = s_k[None]
            sv.append(s_k)
        acc_i = wsc_ref[2, c0] * sv[0] + wsc_ref[2, c0 + 1] * sv[1]
        acc_j = wsc_ref[3, c0] * sv[0] + wsc_ref[3, c0 + 1] * sv[1]
        for k in range(2, 16):
            acc_i = acc_i + wsc_ref[2, c0 + k] * sv[k]
            acc_j = acc_j + wsc_ref[3, c0 + k] * sv[k]
        di_scr[...] = di_scr[...] + acc_i
        dj_scr[...] = dj_scr[...] + acc_j
        return 0

    lax.fori_loop(0, C // 16, pass1, 0)

    bai = wsc_ref[4, 0]
    baj = wsc_ref[4, 1]
    di_scr[...] = jax.nn.sigmoid(di_scr[...] + bai)
    dj_scr[...] = jax.nn.sigmoid(dj_scr[...] + baj)

    pi_scr[...] = jnp.zeros((N, C), f32)
    pjt_scr[...] = jnp.zeros((C, N), f32)

    def pass2(t, _):
        c0 = 16 * t
        sigi = di_scr[...]
        sigj = dj_scr[...]
        pcl = []
        prl = []
        for k in range(16):
            s_k = s_scr[c0 + k]
            pcl.append(jnp.dot(s_k * sigi, ones_col,
                               preferred_element_type=f32))
            prl.append(jnp.dot(ones_row, s_k * sigj,
                               preferred_element_type=f32))
        pcols = jnp.concatenate(pcl, axis=1)
        prows = jnp.concatenate(prl, axis=0)
        ohrows = (iota_8c == c0 + eight_row).astype(f32)
        pi_scr[...] = pi_scr[...] + jnp.dot(pcols, ohrows,
                                            preferred_element_type=f32)
        pjt_scr[...] = pjt_scr[...] + lax.dot_general(
            ohrows, prows, (((0,), (0,)), ((), ())),
            preferred_element_type=f32)
        return 0

    lax.fori_loop(0, C // 16, pass2, 0)

    xo = (jnp.dot(xb, wnx_ref[...], preferred_element_type=f32)
          + jnp.dot(pi_scr[...], wni_ref[...], preferred_element_type=f32)
          + lax.dot_general(pjt_scr[...], wnj_ref[...],
                            (((0,), (0,)), ((), ())),
                            preferred_element_type=f32)
          + bn_ref[...])
    out_ref[0] = jnp.dot(xo, wd_ref[...], preferred_element_type=f32) \
        + bd_ref[...]


def kernel(x, a, e, Ws, bs, Wai, bai, Waj, baj, Wn, bn, We, be, Wd, bd):
    B, N, F = x.shape
    C = Ws.shape[1]
    LBL = Wd.shape[1]
    f32 = jnp.float32

    e2 = e[..., 0]
    et2 = jnp.swapaxes(e2, 1, 2)
    wsi = Ws[:F]
    wsj = Ws[F:2 * F]
    brow = jnp.zeros((C,), f32).at[0].set(bai[0]).at[1].set(baj[0])
    wsc = jnp.stack([Ws[2 * F], Ws[2 * F + 1], Wai[:, 0], Waj[:, 0], brow],
                    axis=0)
    wnx = Wn[:F]
    wni = Wn[F:F + C]
    wnj = Wn[F + C:]

    out = pl.pallas_call(
        _net_body,
        grid=(B,),
        in_specs=[
            pl.BlockSpec((1, N, F), lambda b: (b, 0, 0)),
            pl.BlockSpec((1, N, N), lambda b: (b, 0, 0)),
            pl.BlockSpec((1, N, N), lambda b: (b, 0, 0)),
            pl.BlockSpec((1, N, N), lambda b: (b, 0, 0)),
            pl.BlockSpec((F, C), lambda b: (0, 0)),
            pl.BlockSpec((F, C), lambda b: (0, 0)),
            pl.BlockSpec((1, C), lambda b: (0, 0)),
            pl.BlockSpec(memory_space=pltpu.SMEM),
            pl.BlockSpec((F, F), lambda b: (0, 0)),
            pl.BlockSpec((C, F), lambda b: (0, 0)),
            pl.BlockSpec((C, F), lambda b: (0, 0)),
            pl.BlockSpec((1, F), lambda b: (0, 0)),
            pl.BlockSpec((F, LBL), lambda b: (0, 0)),
            pl.BlockSpec((1, LBL), lambda b: (0, 0)),
        ],
        out_specs=pl.BlockSpec((1, N, LBL), lambda b: (b, 0, 0)),
        out_shape=jax.ShapeDtypeStruct((B, N, LBL), f32),
        scratch_shapes=[
            pltpu.VMEM((C, N, N), f32),
            pltpu.VMEM((C, N), f32),
            pltpu.VMEM((N, C), f32),
            pltpu.VMEM((N, N), f32),
            pltpu.VMEM((N, N), f32),
            pltpu.VMEM((N, C), f32),
            pltpu.VMEM((C, N), f32),
        ],
    )(x, a, e2, et2, wsi, wsj, bs[None], wsc, wnx, wni, wnj,
      bn[None], Wd, bd[None])
    return out


# speedup vs baseline: 2.4393x; 1.0020x over previous
import jax
import jax.numpy as jnp
from jax import lax
from jax.experimental import pallas as pl
from jax.experimental.pallas import tpu as pltpu


def _net_body(x_ref, a_ref, e_ref, et_ref, wsi_ref, wsj_ref, bs_ref,
              wsc_ref, wnx_ref, wni_ref, wnj_ref, bn_ref, wd_ref, bd_ref,
              out_ref, s_scr, vt_scr, u_scr, di_scr, dj_scr, pi_scr, pjt_scr):
    N = a_ref.shape[1]
    C = bs_ref.shape[1]
    f32 = jnp.float32

    xb = x_ref[0]

    u_scr[...] = jnp.dot(xb, wsi_ref[...],
                         preferred_element_type=f32) + bs_ref[...]
    v = jnp.dot(xb, wsj_ref[...], preferred_element_type=f32)
    vt_scr[...] = v.T

    ones_col = jnp.ones((N, 1), f32)
    ones_row = jnp.ones((1, N), f32)
    iota_cc = lax.broadcasted_iota(jnp.int32, (C, 16), 0)
    iota_8c = lax.broadcasted_iota(jnp.int32, (16, C), 1)
    eight_col = lax.broadcasted_iota(jnp.int32, (C, 16), 1)
    eight_row = lax.broadcasted_iota(jnp.int32, (16, C), 0)

    di_scr[...] = jnp.zeros((N, N), f32)
    dj_scr[...] = jnp.zeros((N, N), f32)

    def pass1(c0):
        eb = e_ref[0]
        etb = et_ref[0]
        ab = a_ref[0]
        oh8 = (iota_cc == c0 + eight_col).astype(f32)
        ucols = jnp.dot(u_scr[...], oh8, preferred_element_type=f32)
        sv = []
        for k in range(16):
            ub = jnp.broadcast_to(ucols[:, k:k + 1], (N, N))
            vb = jnp.broadcast_to(vt_scr[pl.ds(c0 + k, 1), :], (N, N))
            s_k = jnp.maximum(ub + vb + wsc_ref[0, c0 + k] * eb
                              + wsc_ref[1, c0 + k] * etb, 0.0) * ab
            s_scr[pl.ds(c0 + k, 1), :, :] = s_k[None]
            sv.append(s_k)
        acc_i = wsc_ref[2, c0] * sv[0] + wsc_ref[2, c0 + 1] * sv[1]
        acc_j = wsc_ref[3, c0] * sv[0] + wsc_ref[3, c0 + 1] * sv[1]
        for k in range(2, 16):
            acc_i = acc_i + wsc_ref[2, c0 + k] * sv[k]
            acc_j = acc_j + wsc_ref[3, c0 + k] * sv[k]
        di_scr[...] = di_scr[...] + acc_i
        dj_scr[...] = dj_scr[...] + acc_j

    pass1(0)
    pass1(16)

    bai = wsc_ref[4, 0]
    baj = wsc_ref[4, 1]
    di_scr[...] = jax.nn.sigmoid(di_scr[...] + bai)
    dj_scr[...] = jax.nn.sigmoid(dj_scr[...] + baj)

    pi_scr[...] = jnp.zeros((N, C), f32)
    pjt_scr[...] = jnp.zeros((C, N), f32)

    def pass2(c0):
        sigi = di_scr[...]
        sigj = dj_scr[...]
        pcl = []
        prl = []
        for k in range(16):
            s_k = s_scr[c0 + k]
            pcl.append(jnp.dot(s_k * sigi, ones_col,
                               preferred_element_type=f32))
            prl.append(jnp.dot(ones_row, s_k * sigj,
                               preferred_element_type=f32))
        pcols = jnp.concatenate(pcl, axis=1)
        prows = jnp.concatenate(prl, axis=0)
        ohrows = (iota_8c == c0 + eight_row).astype(f32)
        pi_scr[...] = pi_scr[...] + jnp.dot(pcols, ohrows,
                                            preferred_element_type=f32)
        pjt_scr[...] = pjt_scr[...] + lax.dot_general(
            ohrows, prows, (((0,), (0,)), ((), ())),
            preferred_element_type=f32)

    pass2(0)
    pass2(16)

    xo = (jnp.dot(xb, wnx_ref[...], preferred_element_type=f32)
          + jnp.dot(pi_scr[...], wni_ref[...], preferred_element_type=f32)
          + lax.dot_general(pjt_scr[...], wnj_ref[...],
                            (((0,), (0,)), ((), ())),
                            preferred_element_type=f32)
          + bn_ref[...])
    out_ref[0] = jnp.dot(xo, wd_ref[...], preferred_element_type=f32) \
        + bd_ref[...]


def kernel(x, a, e, Ws, bs, Wai, bai, Waj, baj, Wn, bn, We, be, Wd, bd):
    B, N, F = x.shape
    C = Ws.shape[1]
    LBL = Wd.shape[1]
    f32 = jnp.float32

    e2 = e[..., 0]
    et2 = jnp.swapaxes(e2, 1, 2)
    wsi = Ws[:F]
    wsj = Ws[F:2 * F]
    brow = jnp.zeros((C,), f32).at[0].set(bai[0]).at[1].set(baj[0])
    wsc = jnp.stack([Ws[2 * F], Ws[2 * F + 1], Wai[:, 0], Waj[:, 0], brow],
                    axis=0)
    wnx = Wn[:F]
    wni = Wn[F:F + C]
    wnj = Wn[F + C:]

    out = pl.pallas_call(
        _net_body,
        grid=(B,),
        in_specs=[
            pl.BlockSpec((1, N, F), lambda b: (b, 0, 0)),
            pl.BlockSpec((1, N, N), lambda b: (b, 0, 0)),
            pl.BlockSpec((1, N, N), lambda b: (b, 0, 0)),
            pl.BlockSpec((1, N, N), lambda b: (b, 0, 0)),
            pl.BlockSpec((F, C), lambda b: (0, 0)),
            pl.BlockSpec((F, C), lambda b: (0, 0)),
            pl.BlockSpec((1, C), lambda b: (0, 0)),
            pl.BlockSpec(memory_space=pltpu.SMEM),
            pl.BlockSpec((F, F), lambda b: (0, 0)),
            pl.BlockSpec((C, F), lambda b: (0, 0)),
            pl.BlockSpec((C, F), lambda b: (0, 0)),
            pl.BlockSpec((1, F), lambda b: (0, 0)),
            pl.BlockSpec((F, LBL), lambda b: (0, 0)),
            pl.BlockSpec((1, LBL), lambda b: (0, 0)),
        ],
        out_specs=pl.BlockSpec((1, N, LBL), lambda b: (b, 0, 0)),
        out_shape=jax.ShapeDtypeStruct((B, N, LBL), f32),
        scratch_shapes=[
            pltpu.VMEM((C, N, N), f32),
            pltpu.VMEM((C, N), f32),
            pltpu.VMEM((N, C), f32),
            pltpu.VMEM((N, N), f32),
            pltpu.VMEM((N, N), f32),
            pltpu.VMEM((N, C), f32),
            pltpu.VMEM((C, N), f32),
        ],
    )(x, a, e2, et2, wsi, wsj, bs[None], wsc, wnx, wni, wnj,
      bn[None], Wd, bd[None])
    return out
